# baseline (speedup 1.0000x reference)
import functools
import math

import jax
import jax.numpy as jnp
from jax.experimental import pallas as pl
from jax.experimental.pallas import tpu as pltpu

_NUM_SLOTS = 8192
_D = 256
_RDIM = 48
_K = 8
_QB = 256
_INV_SQRT = 1.0 / math.sqrt(_RDIM)


def _router_kernel(q_ref, ak_ref, mask_ref, w_ref, idx_ref, val_ref, rk_ref):
    @pl.when(pl.program_id(0) == 0)
    def _():
        rk_ref[...] = jax.lax.dot_general(
            ak_ref[...], w_ref[...],
            (((1,), (1,)), ((), ())),
            preferred_element_type=jnp.float32,
        )

    rq = jax.lax.dot_general(
        q_ref[...], w_ref[...],
        (((1,), (1,)), ((), ())),
        preferred_element_type=jnp.float32,
    )
    s = jax.lax.dot_general(
        rq, rk_ref[...],
        (((1,), (1,)), ((), ())),
        preferred_element_type=jnp.float32,
    )
    s = s * _INV_SQRT + mask_ref[...]

    iota = jax.lax.broadcasted_iota(jnp.int32, s.shape, 1).astype(jnp.float32)
    big = jnp.float32(2.0 * _NUM_SLOTS)
    ninf = jnp.float32(-jnp.inf)
    nch = 16
    cw = _NUM_SLOTS // nch
    ich = [iota[:, c * cw:(c + 1) * cw] for c in range(nch)]

    vals, idxs = [], []
    for _ in range(_K):
        pairs = [(s[:, c * cw:(c + 1) * cw], ich[c]) for c in range(nch)]
        while len(pairs) > 1:
            nxt = []
            for j in range(0, len(pairs), 2):
                (av, ai), (bv, bi) = pairs[j], pairs[j + 1]
                keep = av >= bv
                nxt.append((jnp.where(keep, av, bv), jnp.where(keep, ai, bi)))
            pairs = nxt
        fv, fi = pairs[0]
        m = jnp.max(fv, axis=1, keepdims=True)
        ix = jnp.min(jnp.where(fv == m, fi, big), axis=1, keepdims=True)
        s = jnp.where(iota == ix, ninf, s)
        vals.append(m)
        idxs.append(ix)
    val_ref[...] = jnp.concatenate(vals, axis=1)
    idx_ref[...] = jnp.concatenate(idxs, axis=1).astype(jnp.int32)


@functools.partial(jax.jit, static_argnames=())
def kernel(query, aux_keys, reliability_mask, W):
    b, sq, d = query.shape
    rows = b * sq
    q2 = query.reshape(rows, d)
    mask2 = reliability_mask.reshape(1, _NUM_SLOTS)
    grid = rows // _QB
    idx, val = pl.pallas_call(
        _router_kernel,
        grid=(grid,),
        in_specs=[
            pl.BlockSpec((_QB, d), lambda i: (i, 0)),
            pl.BlockSpec((_NUM_SLOTS, d), lambda i: (0, 0)),
            pl.BlockSpec((1, _NUM_SLOTS), lambda i: (0, 0)),
            pl.BlockSpec((_RDIM, d), lambda i: (0, 0)),
        ],
        out_specs=[
            pl.BlockSpec((_QB, _K), lambda i: (i, 0)),
            pl.BlockSpec((_QB, _K), lambda i: (i, 0)),
        ],
        out_shape=[
            jax.ShapeDtypeStruct((rows, _K), jnp.int32),
            jax.ShapeDtypeStruct((rows, _K), jnp.float32),
        ],
        scratch_shapes=[pltpu.VMEM((_NUM_SLOTS, _RDIM), jnp.float32)],
    )(q2, aux_keys, mask2, W)
    return idx.reshape(b, sq, _K), val.reshape(b, sq, _K)


# speedup vs baseline: 1.3154x; 1.3154x over previous
import functools
import math

import jax
import jax.numpy as jnp
from jax.experimental import pallas as pl
from jax.experimental.pallas import tpu as pltpu

_NUM_SLOTS = 8192
_D = 256
_RDIM = 48
_K = 8
_QB = 512
_INV_SQRT = 1.0 / math.sqrt(_RDIM)


def _router_kernel(q_ref, ak_ref, mask_ref, w_ref, idx_ref, val_ref, rk_ref):
    @pl.when(pl.program_id(0) == 0)
    def _():
        rk_ref[...] = jax.lax.dot_general(
            ak_ref[...], w_ref[...],
            (((1,), (1,)), ((), ())),
            preferred_element_type=jnp.float32,
        )

    rq = jax.lax.dot_general(
        q_ref[...], w_ref[...],
        (((1,), (1,)), ((), ())),
        preferred_element_type=jnp.float32,
    )
    s = jax.lax.dot_general(
        rq, rk_ref[...],
        (((1,), (1,)), ((), ())),
        preferred_element_type=jnp.float32,
    )
    s = s * _INV_SQRT + mask_ref[...]

    iota = jax.lax.broadcasted_iota(jnp.int32, s.shape, 1).astype(jnp.float32)
    big = jnp.float32(2.0 * _NUM_SLOTS)
    vals, idxs = [], []
    for _ in range(_K):
        m = jnp.max(s, axis=1, keepdims=True)
        cand = jnp.where(s == m, iota, big)
        ix = jnp.min(cand, axis=1, keepdims=True)
        s = jnp.where(iota == ix, -jnp.inf, s)
        vals.append(m)
        idxs.append(ix)
    val_ref[...] = jnp.concatenate(vals, axis=1)
    idx_ref[...] = jnp.concatenate(idxs, axis=1).astype(jnp.int32)


@functools.partial(jax.jit, static_argnames=())
def kernel(query, aux_keys, reliability_mask, W):
    b, sq, d = query.shape
    rows = b * sq
    q2 = query.reshape(rows, d)
    mask2 = reliability_mask.reshape(1, _NUM_SLOTS)
    grid = rows // _QB
    idx, val = pl.pallas_call(
        _router_kernel,
        grid=(grid,),
        in_specs=[
            pl.BlockSpec((_QB, d), lambda i: (i, 0)),
            pl.BlockSpec((_NUM_SLOTS, d), lambda i: (0, 0)),
            pl.BlockSpec((1, _NUM_SLOTS), lambda i: (0, 0)),
            pl.BlockSpec((_RDIM, d), lambda i: (0, 0)),
        ],
        out_specs=[
            pl.BlockSpec((_QB, _K), lambda i: (i, 0)),
            pl.BlockSpec((_QB, _K), lambda i: (i, 0)),
        ],
        out_shape=[
            jax.ShapeDtypeStruct((rows, _K), jnp.int32),
            jax.ShapeDtypeStruct((rows, _K), jnp.float32),
        ],
        scratch_shapes=[pltpu.VMEM((_NUM_SLOTS, _RDIM), jnp.float32)],
    )(q2, aux_keys, mask2, W)
    return idx.reshape(b, sq, _K), val.reshape(b, sq, _K)
